# baseline (speedup 1.0000x reference)
import functools

import jax
import jax.numpy as jnp
from jax import lax
from jax.experimental import pallas as pl
from jax.experimental.pallas import tpu as pltpu
from jax.experimental.pallas import tpu_sc as plsc

N = 10000
N_PAD = 10240
E = 320000
D = 128
NC = 2
NS = 16
C = 80
K = E // (NC * NS * C)
P = 5
H = K // P
ROWS_PER_TILE = N_PAD // NS

_mesh = plsc.VectorSubcoreMesh(core_axis_name="c", subcore_axis_name="s")


@functools.partial(
    pl.kernel,
    out_type=jax.ShapeDtypeStruct((NC, N_PAD, 16), jnp.float32),
    mesh=_mesh,
    scratch_types=[
        pltpu.VMEM((K, C), jnp.int32),
        pltpu.VMEM((C, 16), jnp.float32),
        pltpu.VMEM_SHARED((N_PAD, 16), jnp.float32),
    ],
)
def _sc_deg(dst_hbm, zeros_hbm, out_hbm, dst_v, ones_v, acc):
    c = lax.axis_index("c")
    s = lax.axis_index("s")
    tile_id = c * NS + s

    pltpu.sync_copy(dst_hbm.at[tile_id], dst_v)
    for i in range(C):
        ones_v[i, :] = jnp.ones((16,), jnp.float32)

    pltpu.sync_copy(
        zeros_hbm.at[pl.ds(s * ROWS_PER_TILE, ROWS_PER_TILE)],
        acc.at[pl.ds(s * ROWS_PER_TILE, ROWS_PER_TILE)],
    )
    plsc.subcore_barrier()

    def body(j, carry):
        pltpu.sync_copy(ones_v, acc.at[dst_v.at[j]], add=True)
        return carry

    lax.fori_loop(0, K, body, 0)
    plsc.subcore_barrier()

    pltpu.sync_copy(
        acc.at[pl.ds(s * ROWS_PER_TILE, ROWS_PER_TILE)],
        out_hbm.at[c, pl.ds(s * ROWS_PER_TILE, ROWS_PER_TILE)],
    )


@functools.partial(
    pl.kernel,
    out_type=jax.ShapeDtypeStruct((NC, N_PAD, D), jnp.float32),
    mesh=_mesh,
    scratch_types=[
        pltpu.VMEM((H, C), jnp.int32),
        pltpu.VMEM((H, C), jnp.int32),
        pltpu.VMEM((C, D), jnp.float32),
        pltpu.VMEM((C, D), jnp.float32),
        pltpu.SemaphoreType.DMA,
        pltpu.SemaphoreType.DMA,
        pltpu.SemaphoreType.DMA,
        pltpu.SemaphoreType.DMA,
        pltpu.VMEM_SHARED((N_PAD, D), jnp.float32),
    ],
)
def _sc_agg(g_hbm, src_hbm, dst_hbm, zeros_hbm, out_hbm,
            src_v, dst_v, rows0_v, rows1_v, sem0, sem1, ssem0, ssem1, acc):
    c = lax.axis_index("c")
    s = lax.axis_index("s")
    tile_id = c * NS + s

    def drain_scatter(sem):
        pltpu.make_async_copy(g_hbm.at[pl.ds(0, C)], rows0_v, sem).wait()

    pltpu.sync_copy(
        zeros_hbm.at[pl.ds(s * ROWS_PER_TILE, ROWS_PER_TILE)],
        acc.at[pl.ds(s * ROWS_PER_TILE, ROWS_PER_TILE)],
    )
    plsc.subcore_barrier()

    def phase(p, carry):
        pltpu.sync_copy(src_hbm.at[tile_id, p], src_v)
        pltpu.sync_copy(dst_hbm.at[tile_id, p], dst_v)
        pltpu.async_copy(g_hbm.at[src_v.at[0]], rows0_v, sem0)
        pltpu.async_copy(g_hbm.at[src_v.at[1]], rows1_v, sem1)
        pltpu.make_async_copy(g_hbm.at[src_v.at[0]], rows0_v, sem0).wait()
        pltpu.async_copy(rows0_v, acc.at[dst_v.at[0]], ssem0, add=True)
        drain_scatter(ssem0)
        pltpu.async_copy(g_hbm.at[src_v.at[2]], rows0_v, sem0)

        def body(jj, carry2):
            j = 2 * jj + 1
            pltpu.make_async_copy(g_hbm.at[src_v.at[1]], rows1_v, sem1).wait()
            pltpu.async_copy(rows1_v, acc.at[dst_v.at[j]], ssem1, add=True)
            pltpu.make_async_copy(g_hbm.at[src_v.at[0]], rows0_v, sem0).wait()
            pltpu.async_copy(rows0_v, acc.at[dst_v.at[j + 1]], ssem0, add=True)
            drain_scatter(ssem1)
            pltpu.async_copy(g_hbm.at[src_v.at[(j + 2) % H]], rows1_v, sem1)
            drain_scatter(ssem0)
            pltpu.async_copy(g_hbm.at[src_v.at[(j + 3) % H]], rows0_v, sem0)
            return carry2

        lax.fori_loop(0, (H - 1) // 2, body, 0)
        pltpu.make_async_copy(g_hbm.at[src_v.at[0]], rows0_v, sem0).wait()
        pltpu.make_async_copy(g_hbm.at[src_v.at[1]], rows1_v, sem1).wait()
        return carry

    lax.fori_loop(0, P, phase, 0)
    plsc.subcore_barrier()

    pltpu.sync_copy(
        acc.at[pl.ds(s * ROWS_PER_TILE, ROWS_PER_TILE)],
        out_hbm.at[c, pl.ds(s * ROWS_PER_TILE, ROWS_PER_TILE)],
    )


_BN = 1000


def _row_spec(width):
    return pl.BlockSpec((_BN, width), lambda i: (i, 0))


def _full_spec(shape):
    return pl.BlockSpec(shape, lambda i: (0, 0))


def _tc_g_body(x_ref, w_ref, d_ref, o_ref):
    o_ref[...] = d_ref[...] * jnp.dot(
        x_ref[...], w_ref[...], preferred_element_type=jnp.float32)


def _tc_g(x, W, dis):
    return pl.pallas_call(
        _tc_g_body,
        grid=(N // _BN,),
        in_specs=[_row_spec(D), _full_spec((D, D)), _row_spec(1)],
        out_specs=_row_spec(D),
        out_shape=jax.ShapeDtypeStruct((N, D), jnp.float32),
    )(x, W, dis)


def _tc_mid_body(a0_ref, a1_ref, g_ref, d_ref, b_ref, w_ref, o_ref):
    d = d_ref[...]
    h = jnp.tanh(d * (a0_ref[...] + a1_ref[...] + g_ref[...]) + b_ref[...])
    o_ref[...] = d * jnp.dot(h, w_ref[...], preferred_element_type=jnp.float32)


def _tc_mid(a0, a1, g, dis, b, W):
    return pl.pallas_call(
        _tc_mid_body,
        grid=(N // _BN,),
        in_specs=[_row_spec(D), _row_spec(D), _row_spec(D), _row_spec(1),
                  _full_spec((1, D)), _full_spec((D, D))],
        out_specs=_row_spec(D),
        out_shape=jax.ShapeDtypeStruct((N, D), jnp.float32),
    )(a0, a1, g, dis, b, W)


def _tc_out_body(a0_ref, a1_ref, g_ref, d_ref, b_ref, w_ref, br_ref, o_ref):
    h = jnp.tanh(
        d_ref[...] * (a0_ref[...] + a1_ref[...] + g_ref[...]) + b_ref[...])
    o_ref[...] = jnp.dot(
        h, w_ref[...], preferred_element_type=jnp.float32) + br_ref[...]


def _tc_out(a0, a1, g, dis, b, W, br):
    out_w = W.shape[1]
    return pl.pallas_call(
        _tc_out_body,
        grid=(N // _BN,),
        in_specs=[_row_spec(D), _row_spec(D), _row_spec(D), _row_spec(1),
                  _full_spec((1, D)), _full_spec((D, out_w)),
                  _full_spec((1, out_w))],
        out_specs=_row_spec(out_w),
        out_shape=jax.ShapeDtypeStruct((N, out_w), jnp.float32),
    )(a0, a1, g, dis, b, W, br)


def kernel(x, edge_index, W1, b1, W2, b2, Wr, br):
    src = edge_index[0].astype(jnp.int32).reshape(NC * NS, P, H, C)
    dst = edge_index[1].astype(jnp.int32).reshape(NC * NS, P, H, C)
    dst_deg = edge_index[1].astype(jnp.int32).reshape(NC * NS, K, C)
    zeros = jnp.zeros((N_PAD, D), jnp.float32)
    zeros16 = jnp.zeros((N_PAD, 16), jnp.float32)

    degp = _sc_deg(dst_deg, zeros16)
    deg = degp[0, :N, 0] + degp[1, :N, 0] + 1.0
    dis = lax.rsqrt(deg).reshape(N, 1)

    g1 = _tc_g(x, W1, dis)
    a1 = _sc_agg(g1, src, dst, zeros)
    g2 = _tc_mid(a1[0, :N], a1[1, :N], g1, dis, b1.reshape(1, D), W2)
    a2 = _sc_agg(g2, src, dst, zeros)
    return _tc_out(a2[0, :N], a2[1, :N], g2, dis, b2.reshape(1, D), Wr,
                   br.reshape(1, -1))


# speedup vs baseline: 1.3476x; 1.3476x over previous
import functools

import jax
import jax.numpy as jnp
from jax import lax
from jax.experimental import pallas as pl
from jax.experimental.pallas import tpu as pltpu
from jax.experimental.pallas import tpu_sc as plsc

N = 10000
N_PAD = 10240
E = 320000
D = 128
NC = 2
NS = 16
C = 125
K = E // (NC * NS * C)
P = 2
H = K // P
ROWS_PER_TILE = N_PAD // NS

_mesh = plsc.VectorSubcoreMesh(core_axis_name="c", subcore_axis_name="s")


@functools.partial(
    pl.kernel,
    out_type=jax.ShapeDtypeStruct((NC, N_PAD, 16), jnp.float32),
    mesh=_mesh,
    scratch_types=[
        pltpu.VMEM((K, C), jnp.int32),
        pltpu.VMEM((C, 16), jnp.float32),
        pltpu.VMEM_SHARED((N_PAD, 16), jnp.float32),
    ],
)
def _sc_deg(dst_hbm, zeros_hbm, out_hbm, dst_v, ones_v, acc):
    c = lax.axis_index("c")
    s = lax.axis_index("s")
    tile_id = c * NS + s

    pltpu.sync_copy(dst_hbm.at[tile_id], dst_v)
    for i in range(C):
        ones_v[i, :] = jnp.ones((16,), jnp.float32)

    pltpu.sync_copy(
        zeros_hbm.at[pl.ds(s * ROWS_PER_TILE, ROWS_PER_TILE)],
        acc.at[pl.ds(s * ROWS_PER_TILE, ROWS_PER_TILE)],
    )
    plsc.subcore_barrier()

    def body(j, carry):
        pltpu.sync_copy(ones_v, acc.at[dst_v.at[j]], add=True)
        return carry

    lax.fori_loop(0, K, body, 0)
    plsc.subcore_barrier()

    pltpu.sync_copy(
        acc.at[pl.ds(s * ROWS_PER_TILE, ROWS_PER_TILE)],
        out_hbm.at[c, pl.ds(s * ROWS_PER_TILE, ROWS_PER_TILE)],
    )


@functools.partial(
    pl.kernel,
    out_type=jax.ShapeDtypeStruct((NC, N_PAD, D), jnp.float32),
    mesh=_mesh,
    scratch_types=[
        pltpu.VMEM((H, C), jnp.int32),
        pltpu.VMEM((H, C), jnp.int32),
        pltpu.VMEM((C, D), jnp.float32),
        pltpu.VMEM((C, D), jnp.float32),
        pltpu.SemaphoreType.DMA,
        pltpu.SemaphoreType.DMA,
        pltpu.VMEM_SHARED((N_PAD, D), jnp.float32),
    ],
)
def _sc_agg(g_hbm, src_hbm, dst_hbm, zeros_hbm, out_hbm,
            src_v, dst_v, rows0_v, rows1_v, sem0, sem1, acc):
    c = lax.axis_index("c")
    s = lax.axis_index("s")
    tile_id = c * NS + s

    pltpu.sync_copy(
        zeros_hbm.at[pl.ds(s * ROWS_PER_TILE, ROWS_PER_TILE)],
        acc.at[pl.ds(s * ROWS_PER_TILE, ROWS_PER_TILE)],
    )
    plsc.subcore_barrier()

    def phase(p, carry):
        pltpu.sync_copy(src_hbm.at[tile_id, p], src_v)
        pltpu.sync_copy(dst_hbm.at[tile_id, p], dst_v)
        pltpu.async_copy(g_hbm.at[src_v.at[0]], rows0_v, sem0)
        pltpu.async_copy(g_hbm.at[src_v.at[1]], rows1_v, sem1)

        def body(jj, carry2):
            j = 2 * jj
            pltpu.make_async_copy(g_hbm.at[src_v.at[0]], rows0_v, sem0).wait()
            pltpu.sync_copy(rows0_v, acc.at[dst_v.at[j]], add=True)
            pltpu.async_copy(g_hbm.at[src_v.at[(j + 2) % H]], rows0_v, sem0)
            pltpu.make_async_copy(g_hbm.at[src_v.at[1]], rows1_v, sem1).wait()
            pltpu.sync_copy(rows1_v, acc.at[dst_v.at[j + 1]], add=True)
            pltpu.async_copy(g_hbm.at[src_v.at[(j + 3) % H]], rows1_v, sem1)
            return carry2

        lax.fori_loop(0, H // 2, body, 0)
        pltpu.make_async_copy(g_hbm.at[src_v.at[0]], rows0_v, sem0).wait()
        pltpu.make_async_copy(g_hbm.at[src_v.at[1]], rows1_v, sem1).wait()
        return carry

    lax.fori_loop(0, P, phase, 0)
    plsc.subcore_barrier()

    pltpu.sync_copy(
        acc.at[pl.ds(s * ROWS_PER_TILE, ROWS_PER_TILE)],
        out_hbm.at[c, pl.ds(s * ROWS_PER_TILE, ROWS_PER_TILE)],
    )


_BN = 1000


def _row_spec(width):
    return pl.BlockSpec((_BN, width), lambda i: (i, 0))


def _full_spec(shape):
    return pl.BlockSpec(shape, lambda i: (0, 0))


def _part_spec(width):
    return [pl.BlockSpec((1, _BN, width), lambda i: (0, i, 0)),
            pl.BlockSpec((1, _BN, width), lambda i: (1, i, 0))]


def _tc_g_body(d0_ref, d1_ref, x_ref, w_ref, g_ref, dis_ref):
    deg = d0_ref[0, :, 0:1] + d1_ref[0, :, 0:1] + 1.0
    dis = lax.rsqrt(deg)
    dis_ref[...] = dis
    g_ref[...] = dis * jnp.dot(
        x_ref[...], w_ref[...], preferred_element_type=jnp.float32)


def _tc_g(degp, x, W):
    return pl.pallas_call(
        _tc_g_body,
        grid=(N // _BN,),
        in_specs=_part_spec(16) + [_row_spec(D), _full_spec((D, D))],
        out_specs=[_row_spec(D), _row_spec(1)],
        out_shape=[jax.ShapeDtypeStruct((N, D), jnp.float32),
                   jax.ShapeDtypeStruct((N, 1), jnp.float32)],
    )(degp, degp, x, W)


def _tc_mid_body(a0_ref, a1_ref, g_ref, d_ref, b_ref, w_ref, o_ref):
    d = d_ref[...]
    h = jnp.tanh(
        d * (a0_ref[0] + a1_ref[0] + g_ref[...]) + b_ref[...])
    o_ref[...] = d * jnp.dot(h, w_ref[...], preferred_element_type=jnp.float32)


def _tc_mid(aggp, g, dis, b, W):
    return pl.pallas_call(
        _tc_mid_body,
        grid=(N // _BN,),
        in_specs=_part_spec(D) + [_row_spec(D), _row_spec(1),
                  _full_spec((1, D)), _full_spec((D, D))],
        out_specs=_row_spec(D),
        out_shape=jax.ShapeDtypeStruct((N, D), jnp.float32),
    )(aggp, aggp, g, dis, b, W)


def _tc_out_body(a0_ref, a1_ref, g_ref, d_ref, b_ref, w_ref, br_ref, o_ref):
    h = jnp.tanh(
        d_ref[...] * (a0_ref[0] + a1_ref[0] + g_ref[...]) + b_ref[...])
    o_ref[...] = jnp.dot(
        h, w_ref[...], preferred_element_type=jnp.float32) + br_ref[...]


def _tc_out(aggp, g, dis, b, W, br):
    out_w = W.shape[1]
    return pl.pallas_call(
        _tc_out_body,
        grid=(N // _BN,),
        in_specs=_part_spec(D) + [_row_spec(D), _row_spec(1),
                  _full_spec((1, D)), _full_spec((D, out_w)),
                  _full_spec((1, out_w))],
        out_specs=_row_spec(out_w),
        out_shape=jax.ShapeDtypeStruct((N, out_w), jnp.float32),
    )(aggp, aggp, g, dis, b, W, br)


def kernel(x, edge_index, W1, b1, W2, b2, Wr, br):
    src = edge_index[0].astype(jnp.int32).reshape(NC * NS, P, H, C)
    dst = edge_index[1].astype(jnp.int32).reshape(NC * NS, P, H, C)
    dst_deg = edge_index[1].astype(jnp.int32).reshape(NC * NS, K, C)
    zeros = jnp.zeros((N_PAD, D), jnp.float32)
    zeros16 = jnp.zeros((N_PAD, 16), jnp.float32)

    degp = _sc_deg(dst_deg, zeros16)
    g1, dis = _tc_g(degp, x, W1)
    a1 = _sc_agg(g1, src, dst, zeros)
    g2 = _tc_mid(a1, g1, dis, b1.reshape(1, D), W2)
    a2 = _sc_agg(g2, src, dst, zeros)
    return _tc_out(a2, g2, dis, b2.reshape(1, D), Wr, br.reshape(1, -1))
